# baseline (speedup 1.0000x reference)
import functools

import jax
import jax.numpy as jnp
from jax import lax
from jax.experimental import pallas as pl
from jax.experimental.pallas import tpu as pltpu
from jax.experimental.pallas import tpu_sc as plsc

_MAX_VALENCE = 6
_NUM_TYPES = 4
_VOCAB = _MAX_VALENCE ** _NUM_TYPES
_EMBED = 64
_BATCH = 16384
_ATOMS = 50
_B = _BATCH * _ATOMS

_NC = 2
_NS = 16
_NW = _NC * _NS
_HALVES = 8
_BATCH_H = _BATCH // _HALVES
_BATCH_PER_W = _BATCH_H // _NW
_CHUNK_B = 8
_CHUNK = _CHUNK_B * _ATOMS
_N_CHUNKS = _BATCH_PER_W // _CHUNK_B
_GATHERS = (128, 128, 128, 16)


def _make_kernel():
  mesh = plsc.VectorSubcoreMesh(core_axis_name="c", subcore_axis_name="s")

  @functools.partial(
      pl.kernel,
      mesh=mesh,
      compiler_params=pltpu.CompilerParams(use_tc_tiling_on_sc=True),
      out_type=jax.ShapeDtypeStruct((_BATCH_H, _ATOMS, _EMBED), jnp.float32),
      scratch_types=[
          pltpu.VMEM((_CHUNK,), jnp.int32),
          pltpu.VMEM((_CHUNK,), jnp.int32),
          pltpu.VMEM((_CHUNK,), jnp.int32),
          pltpu.VMEM((_CHUNK,), jnp.int32),
          pltpu.VMEM((_CHUNK, _EMBED), jnp.float32),
          pltpu.VMEM((_CHUNK, _EMBED), jnp.float32),
          pltpu.VMEM_SHARED((_VOCAB, _EMBED), jnp.float32),
          pltpu.SemaphoreType.DMA,
          pltpu.SemaphoreType.DMA,
          pltpu.SemaphoreType.DMA,
          pltpu.SemaphoreType.DMA,
          pltpu.SemaphoreType.DMA,
          pltpu.SemaphoreType.DMA,
      ],
  )
  def lookup(val_hbm, table_hbm, out_hbm,
             val0, val1, idx0, idx1, rows0, rows1, table_sh,
             sv0, sv1, sg0, sg1, sw0, sw1):
    sid = lax.axis_index("s")
    wid = sid * _NC + lax.axis_index("c")

    @pl.when(sid == 0)
    def _stage_table():
      pltpu.sync_copy(table_hbm, table_sh)

    plsc.subcore_barrier()

    vals = (val0, val1)
    idxs = (idx0, idx1)
    rows = (rows0, rows1)
    sv = (sv0, sv1)
    sg = (sg0, sg1)
    sw = (sw0, sw1)

    def vin(c, b):
      off = (wid * _BATCH_PER_W + c * _CHUNK_B) * _ATOMS
      return pltpu.make_async_copy(val_hbm.at[pl.ds(off, _CHUNK)], vals[b],
                                   sv[b])

    def wouts(c, b):
      b0 = wid * _BATCH_PER_W + c * _CHUNK_B
      return [
          pltpu.make_async_copy(rows[b].at[pl.ds(i * _ATOMS, _ATOMS)],
                                out_hbm.at[b0 + i], sw[b])
          for i in range(_CHUNK_B)
      ]

    def gths(b):
      copies = []
      off = 0
      for n in _GATHERS:
        copies.append(pltpu.make_async_copy(
            table_sh.at[idxs[b].at[pl.ds(off, n)]],
            rows[b].at[pl.ds(off, n)],
            sg[b]))
        off += n
      return copies

    vin(0, 0).start()
    vin(1, 1).start()

    def pair_body(i, carry):
      for b in range(2):
        c = 2 * i + b
        vin(c, b).wait()
        for g in range(_CHUNK // 16):
          gb = g * 16
          w = vals[b][pl.ds(gb, 16)]
          d0 = w & 255
          d1 = (w >> 8) & 255
          d2 = (w >> 16) & 255
          d3 = w >> 24
          idxs[b][pl.ds(gb, 16)] = d0 + d1 * 6 + d2 * 36 + d3 * 216

        @pl.when(c + 2 < _N_CHUNKS)
        def _prefetch():
          vin(c + 2, b).start()

        @pl.when(c >= 2)
        def _drain_prev_write():
          for w_ in wouts(c - 2, b):
            w_.wait()

        for g_ in gths(b):
          g_.start()
        for g_ in gths(b):
          g_.wait()
        for w_ in wouts(c, b):
          w_.start()
      return carry

    lax.fori_loop(0, _N_CHUNKS // 2, pair_body, 0)
    for w_ in wouts(_N_CHUNKS - 2, 0):
      w_.wait()
    for w_ in wouts(_N_CHUNKS - 1, 1):
      w_.wait()

  return lookup


_LOOKUP = _make_kernel()

_TB = 128
_NTB = _BATCH_H // _TB


def _transpose_body(x_ref, o_ref):
  for a in range(_ATOMS):
    o_ref[a] = jnp.transpose(x_ref[:, a, :], (1, 0))


def _transpose_body_alias(x_ref, prev_ref, o_ref):
  del prev_ref
  for a in range(_ATOMS):
    o_ref[a] = jnp.transpose(x_ref[:, a, :], (1, 0))


def _make_t2(part):
  off = part * _NTB
  if part == 0:
    return pl.pallas_call(
        _transpose_body,
        grid=(_NTB,),
        in_specs=[pl.BlockSpec((_TB, _ATOMS, _EMBED), lambda i: (i, 0, 0))],
        out_specs=pl.BlockSpec((_ATOMS, _EMBED, _TB), lambda i: (0, 0, i)),
        out_shape=jax.ShapeDtypeStruct((_ATOMS, _EMBED, _BATCH), jnp.float32),
    )
  return pl.pallas_call(
      _transpose_body_alias,
      grid=(_NTB,),
      in_specs=[
          pl.BlockSpec((_TB, _ATOMS, _EMBED), lambda i: (i, 0, 0)),
          pl.BlockSpec(memory_space=pl.ANY),
      ],
      out_specs=pl.BlockSpec((_ATOMS, _EMBED, _TB),
                             lambda i, off=off: (0, 0, i + off)),
      out_shape=jax.ShapeDtypeStruct((_ATOMS, _EMBED, _BATCH), jnp.float32),
      input_output_aliases={1: 0},
  )


_T2 = [_make_t2(p) for p in range(_HALVES)]


def kernel(valences, embed_table, device):
  val_packed = lax.bitcast_convert_type(
      valences.reshape(_B, _NUM_TYPES).astype(jnp.int8), jnp.int32)
  n = _BATCH_H * _ATOMS
  parts = [_LOOKUP(val_packed[p * n:(p + 1) * n], embed_table)
           for p in range(_HALVES)]
  t = _T2[0](parts[0])
  for p in range(1, _HALVES):
    t = _T2[p](parts[p], t)
  return jnp.transpose(t, (2, 0, 1))


# speedup vs baseline: 1.0100x; 1.0100x over previous
import functools

import jax
import jax.numpy as jnp
from jax import lax
from jax.experimental import pallas as pl
from jax.experimental.pallas import tpu as pltpu
from jax.experimental.pallas import tpu_sc as plsc

_MAX_VALENCE = 6
_NUM_TYPES = 4
_VOCAB = _MAX_VALENCE ** _NUM_TYPES
_EMBED = 64
_BATCH = 16384
_ATOMS = 50
_B = _BATCH * _ATOMS

_NC = 2
_NS = 16
_NW = _NC * _NS
_HALVES = 4
_BATCH_H = _BATCH // _HALVES
_BATCH_PER_W = _BATCH_H // _NW
_CHUNK_B = 8
_CHUNK = _CHUNK_B * _ATOMS
_N_CHUNKS = _BATCH_PER_W // _CHUNK_B
_GATHERS = (128, 128, 128, 16)


def _make_kernel():
  mesh = plsc.VectorSubcoreMesh(core_axis_name="c", subcore_axis_name="s")

  @functools.partial(
      pl.kernel,
      mesh=mesh,
      compiler_params=pltpu.CompilerParams(use_tc_tiling_on_sc=True),
      out_type=jax.ShapeDtypeStruct((_BATCH_H, _ATOMS, _EMBED), jnp.float32),
      scratch_types=[
          pltpu.VMEM((_CHUNK,), jnp.int32),
          pltpu.VMEM((_CHUNK,), jnp.int32),
          pltpu.VMEM((_CHUNK,), jnp.int32),
          pltpu.VMEM((_CHUNK,), jnp.int32),
          pltpu.VMEM((_CHUNK, _EMBED), jnp.float32),
          pltpu.VMEM((_CHUNK, _EMBED), jnp.float32),
          pltpu.VMEM_SHARED((_VOCAB, _EMBED), jnp.float32),
          pltpu.SemaphoreType.DMA,
          pltpu.SemaphoreType.DMA,
          pltpu.SemaphoreType.DMA,
          pltpu.SemaphoreType.DMA,
          pltpu.SemaphoreType.DMA,
          pltpu.SemaphoreType.DMA,
      ],
  )
  def lookup(val_hbm, table_hbm, out_hbm,
             val0, val1, idx0, idx1, rows0, rows1, table_sh,
             sv0, sv1, sg0, sg1, sw0, sw1):
    sid = lax.axis_index("s")
    wid = sid * _NC + lax.axis_index("c")

    @pl.when(sid == 0)
    def _stage_table():
      pltpu.sync_copy(table_hbm, table_sh)

    plsc.subcore_barrier()

    vals = (val0, val1)
    idxs = (idx0, idx1)
    rows = (rows0, rows1)
    sv = (sv0, sv1)
    sg = (sg0, sg1)
    sw = (sw0, sw1)

    def vin(c, b):
      off = (wid * _BATCH_PER_W + c * _CHUNK_B) * _ATOMS
      return pltpu.make_async_copy(val_hbm.at[pl.ds(off, _CHUNK)], vals[b],
                                   sv[b])

    def wouts(c, b):
      b0 = wid * _BATCH_PER_W + c * _CHUNK_B
      return [
          pltpu.make_async_copy(rows[b].at[pl.ds(i * _ATOMS, _ATOMS)],
                                out_hbm.at[b0 + i], sw[b])
          for i in range(_CHUNK_B)
      ]

    def gths(b):
      copies = []
      off = 0
      for n in _GATHERS:
        copies.append(pltpu.make_async_copy(
            table_sh.at[idxs[b].at[pl.ds(off, n)]],
            rows[b].at[pl.ds(off, n)],
            sg[b]))
        off += n
      return copies

    vin(0, 0).start()
    vin(1, 1).start()

    def pair_body(i, carry):
      for b in range(2):
        c = 2 * i + b
        vin(c, b).wait()
        for g in range(_CHUNK // 16):
          gb = g * 16
          w = vals[b][pl.ds(gb, 16)]
          d0 = w & 255
          d1 = (w >> 8) & 255
          d2 = (w >> 16) & 255
          d3 = w >> 24
          idxs[b][pl.ds(gb, 16)] = d0 + d1 * 6 + d2 * 36 + d3 * 216

        @pl.when(c + 2 < _N_CHUNKS)
        def _prefetch():
          vin(c + 2, b).start()

        @pl.when(c >= 2)
        def _drain_prev_write():
          for w_ in wouts(c - 2, b):
            w_.wait()

        for g_ in gths(b):
          g_.start()
        for g_ in gths(b):
          g_.wait()
        for w_ in wouts(c, b):
          w_.start()
      return carry

    lax.fori_loop(0, _N_CHUNKS // 2, pair_body, 0)
    for w_ in wouts(_N_CHUNKS - 2, 0):
      w_.wait()
    for w_ in wouts(_N_CHUNKS - 1, 1):
      w_.wait()

  return lookup


_LOOKUP = _make_kernel()

_TB = 128
_NTB = _BATCH_H // _TB


def _transpose_body(x_ref, o_ref):
  for a in range(_ATOMS):
    o_ref[a] = jnp.transpose(x_ref[:, a, :], (1, 0))


def _transpose_body_alias(x_ref, prev_ref, o_ref):
  del prev_ref
  for a in range(_ATOMS):
    o_ref[a] = jnp.transpose(x_ref[:, a, :], (1, 0))


def _make_t2(part):
  off = part * _NTB
  if part == 0:
    return pl.pallas_call(
        _transpose_body,
        grid=(_NTB,),
        in_specs=[pl.BlockSpec((_TB, _ATOMS, _EMBED), lambda i: (i, 0, 0))],
        out_specs=pl.BlockSpec((_ATOMS, _EMBED, _TB), lambda i: (0, 0, i)),
        out_shape=jax.ShapeDtypeStruct((_ATOMS, _EMBED, _BATCH), jnp.float32),
    )
  return pl.pallas_call(
      _transpose_body_alias,
      grid=(_NTB,),
      in_specs=[
          pl.BlockSpec((_TB, _ATOMS, _EMBED), lambda i: (i, 0, 0)),
          pl.BlockSpec(memory_space=pl.ANY),
      ],
      out_specs=pl.BlockSpec((_ATOMS, _EMBED, _TB),
                             lambda i, off=off: (0, 0, i + off)),
      out_shape=jax.ShapeDtypeStruct((_ATOMS, _EMBED, _BATCH), jnp.float32),
      input_output_aliases={1: 0},
  )


_T2 = [_make_t2(p) for p in range(_HALVES)]


def kernel(valences, embed_table, device):
  val_packed = lax.bitcast_convert_type(
      valences.reshape(_B, _NUM_TYPES).astype(jnp.int8), jnp.int32)
  n = _BATCH_H * _ATOMS
  parts = [_LOOKUP(val_packed[p * n:(p + 1) * n], embed_table)
           for p in range(_HALVES)]
  t = _T2[0](parts[0])
  for p in range(1, _HALVES):
    t = _T2[p](parts[p], t)
  return jnp.transpose(t, (2, 0, 1))
